# baseline (speedup 1.0000x reference)
import functools

import jax
import jax.numpy as jnp
from jax import lax
from jax.experimental import pallas as pl
from jax.experimental.pallas import tpu as pltpu
from jax.experimental.pallas import tpu_sc as plsc

NUM_CORES = 2
NUM_SUBCORES = 16
NUM_WORKERS = NUM_CORES * NUM_SUBCORES
CHUNK = 256


def kernel(labels, embedding_table):
    B = labels.shape[0]
    V, D = embedding_table.shape
    b_per_w = B // NUM_WORKERS
    n_chunks = b_per_w // CHUNK

    labels = labels.astype(jnp.int32)
    packed = embedding_table[: (V - 1)].reshape((V - 1) // 2, 2 * D)
    pair_idx = labels >> 1
    gidx = (
        (jnp.arange(B, dtype=jnp.int32)[:, None] & (CHUNK - 1)) * (2 * D)
        + (labels[:, None] & 1) * D
        + jnp.arange(D, dtype=jnp.int32)[None, :]
    ).reshape(-1)

    mesh = plsc.VectorSubcoreMesh(core_axis_name="c", subcore_axis_name="s")

    @functools.partial(
        pl.kernel,
        mesh=mesh,
        out_type=jax.ShapeDtypeStruct((B * D,), jnp.float32),
        scratch_types=[
            pltpu.VMEM((CHUNK,), jnp.int32),
            pltpu.VMEM((CHUNK * D,), jnp.int32),
            pltpu.VMEM((CHUNK, 2 * D), jnp.float32),
            pltpu.VMEM((CHUNK * D,), jnp.float32),
            pltpu.SemaphoreType.DMA,
        ],
        compiler_params=pltpu.CompilerParams(needs_layout_passes=False),
    )
    def emb(pidx_hbm, gidx_hbm, table_hbm, out_hbm, pidx_v, gidx_v, prows, orows, sem):
        wid = lax.axis_index("s") * NUM_CORES + lax.axis_index("c")
        for h in range(n_chunks):
            base = wid * b_per_w + h * CHUNK
            pltpu.sync_copy(pidx_hbm.at[pl.ds(base, CHUNK)], pidx_v)
            pltpu.sync_copy(gidx_hbm.at[pl.ds(base * D, CHUNK * D)], gidx_v)
            pltpu.async_copy(table_hbm.at[pidx_v], prows, sem).wait()

            def body(g, carry):
                idx = gidx_v[pl.ds(g * 16, 16)]
                vals = plsc.load_gather(prows, [idx >> 7, idx & 127])
                orows[pl.ds(g * 16, 16)] = vals
                return carry

            lax.fori_loop(0, CHUNK * D // 16, body, 0)
            pltpu.sync_copy(orows, out_hbm.at[pl.ds(base * D, CHUNK * D)])

    out_flat = emb(pair_idx, gidx, packed)
    return out_flat.reshape(B, D)
